# Initial kernel scaffold; baseline (speedup 1.0000x reference)
import jax
import jax.numpy as jnp
from jax.experimental import pallas as pl


def kernel(sequence, token_0, token_mask, pe_weight, embeddings):
    raise NotImplementedError("write your pallas kernel here")


# speedup vs baseline: 2.0427x; 2.0427x over previous
import jax
import jax.numpy as jnp
from jax import lax
from jax.experimental import pallas as pl
from jax.experimental.pallas import tpu as pltpu
from jax.experimental.pallas import tpu_sc as plsc

_EMBED = 128
_MAX_LEN = 200
_NUM_ITEMS = 100000
_BATCH = 1024

_NC, _NS = 2, 16
_NW = _NC * _NS
_ROWS = _BATCH * _MAX_LEN
_CHUNK = 128
_NCHUNK = _ROWS // _CHUNK
_CPW = _NCHUNK // _NW
_NV = _EMBED // 16


def _body(ridx_hbm, sidx_hbm, t0_hbm, tm_hbm, pe_hbm, emb_hbm, out_hbm,
          ridx_v, sidx_v, pe2_v, t0_v, tm_v, rv, gsem):
    w = lax.axis_index("s") * _NC + lax.axis_index("c")
    base = w * _CPW
    pltpu.sync_copy(ridx_hbm.at[pl.ds(base, _CPW)], ridx_v)
    pltpu.sync_copy(sidx_hbm.at[pl.ds(base, _CPW)], sidx_v)
    pltpu.sync_copy(pe_hbm, pe2_v.at[pl.ds(0, _MAX_LEN)])
    pltpu.sync_copy(pe_hbm, pe2_v.at[pl.ds(_MAX_LEN, _MAX_LEN)])
    pltpu.sync_copy(t0_hbm, t0_v)
    pltpu.sync_copy(tm_hbm, tm_v)
    t0r = [t0_v[0, pl.ds(16 * j, 16)] for j in range(_NV)]
    tmr = [tm_v[0, pl.ds(16 * j, 16)] for j in range(_NV)]

    @pl.loop(0, _CPW)
    def _chunk(c):
        pltpu.async_copy(emb_hbm.at[sidx_v.at[c]], rv, gsem).wait()
        tc = (c * _CHUNK) % _MAX_LEN

        @pl.loop(0, _CHUNK)
        def _row(l):
            iv = plsc.load_gather(
                ridx_v,
                [jnp.full((16,), c, jnp.int32), jnp.full((16,), l, jnp.int32)])
            f0 = iv == 0
            fm = iv == _NUM_ITEMS + 1
            t = tc + l
            for j in range(_NV):
                v = rv[l, pl.ds(16 * j, 16)]
                p = pe2_v[t, pl.ds(16 * j, 16)]
                v = jnp.where(f0, t0r[j], jnp.where(fm, tmr[j], v)) + p
                rv[l, pl.ds(16 * j, 16)] = v

        pltpu.sync_copy(rv, out_hbm.at[pl.ds((base + c) * _CHUNK, _CHUNK)])


def kernel(sequence, token_0, token_mask, pe_weight, embeddings):
    seq = sequence.reshape(_NCHUNK, _CHUNK).astype(jnp.int32)
    sidx = jnp.clip(seq - 1, 0, _NUM_ITEMS - 1)
    mesh = plsc.VectorSubcoreMesh(core_axis_name="c", subcore_axis_name="s",
                                  num_cores=_NC, num_subcores=_NS)
    out = pl.kernel(
        _body,
        out_type=jax.ShapeDtypeStruct((_ROWS, _EMBED), jnp.float32),
        mesh=mesh,
        scratch_types=[
            pltpu.VMEM((_CPW, _CHUNK), jnp.int32),
            pltpu.VMEM((_CPW, _CHUNK), jnp.int32),
            pltpu.VMEM((2 * _MAX_LEN, _EMBED), jnp.float32),
            pltpu.VMEM((1, _EMBED), jnp.float32),
            pltpu.VMEM((1, _EMBED), jnp.float32),
            pltpu.VMEM((_CHUNK, _EMBED), jnp.float32),
            pltpu.SemaphoreType.DMA,
        ],
        compiler_params=pltpu.CompilerParams(use_tc_tiling_on_sc=False,
                                             needs_layout_passes=False),
    )(seq, sidx, token_0, token_mask, pe_weight, embeddings)
    x = out.reshape(_BATCH, _MAX_LEN, _EMBED)
    return (x, embeddings)


# speedup vs baseline: 2.9559x; 1.4470x over previous
import jax
import jax.numpy as jnp
from jax import lax
from jax.experimental import pallas as pl
from jax.experimental.pallas import tpu as pltpu
from jax.experimental.pallas import tpu_sc as plsc

_EMBED = 128
_MAX_LEN = 200
_NUM_ITEMS = 100000
_BATCH = 1024

_NC, _NS = 2, 16
_NW = _NC * _NS
_ROWS = _BATCH * _MAX_LEN
_CHUNK = 128
_NCHUNK = _ROWS // _CHUNK
_CPW = _NCHUNK // _NW
_NV = _EMBED // 16
_PE2 = _MAX_LEN + _CHUNK


def _body(ridx_hbm, sidx_hbm, t0_hbm, tm_hbm, pe_hbm, emb_hbm, out_hbm,
          ridx_v, sidx_v, pe2_v, t0_v, tm_v, rv, ob,
          gsem0, gsem1, osem0, osem1):
    gsems = (gsem0, gsem1)
    osems = (osem0, osem1)
    w = lax.axis_index("s") * _NC + lax.axis_index("c")
    base = w * _CPW
    pltpu.sync_copy(ridx_hbm.at[pl.ds(base, _CPW)], ridx_v)
    pltpu.sync_copy(sidx_hbm.at[pl.ds(base, _CPW)], sidx_v)
    pltpu.sync_copy(pe_hbm, pe2_v.at[pl.ds(0, _MAX_LEN)])
    pltpu.sync_copy(pe_hbm.at[pl.ds(0, _CHUNK)], pe2_v.at[pl.ds(_MAX_LEN, _CHUNK)])
    pltpu.sync_copy(t0_hbm, t0_v)
    pltpu.sync_copy(tm_hbm, tm_v)
    t0r = [t0_v[0, pl.ds(16 * j, 16)] for j in range(_NV)]
    tmr = [tm_v[0, pl.ds(16 * j, 16)] for j in range(_NV)]

    def compute(c, b):
        rvb, obb = rv.at[b], ob.at[b]
        tc = (c * _CHUNK) % _MAX_LEN
        m = None
        for k in range(_NV):
            iv = ridx_v[c, pl.ds(16 * k, 16)]
            mk = (iv == 0) | (iv == _NUM_ITEMS + 1)
            m = mk if m is None else (m | mk)
        any_spec = jnp.any(m)

        @pl.when(jnp.logical_not(any_spec))
        def _fast():
            @pl.loop(0, _CHUNK)
            def _row(l):
                t = tc + l
                for j in range(_NV):
                    obb[l, pl.ds(16 * j, 16)] = (
                        rvb[l, pl.ds(16 * j, 16)] + pe2_v[t, pl.ds(16 * j, 16)])

        @pl.when(any_spec)
        def _slow():
            @pl.loop(0, _CHUNK)
            def _row(l):
                iv = plsc.load_gather(
                    ridx_v,
                    [jnp.full((16,), c, jnp.int32), jnp.full((16,), l, jnp.int32)])
                f0 = iv == 0
                fm = iv == _NUM_ITEMS + 1
                t = tc + l
                for j in range(_NV):
                    v = rvb[l, pl.ds(16 * j, 16)]
                    v = jnp.where(f0, t0r[j], jnp.where(fm, tmr[j], v))
                    obb[l, pl.ds(16 * j, 16)] = v + pe2_v[t, pl.ds(16 * j, 16)]

    for b in range(2):
        pltpu.async_copy(emb_hbm.at[sidx_v.at[b]], rv.at[b], gsems[b])

    @pl.loop(0, _CPW, step=2)
    def _pair(c0):
        for b in range(2):
            c = c0 + b
            pltpu.make_async_copy(emb_hbm.at[sidx_v.at[c]], rv.at[b],
                                  gsems[b]).wait()

            @pl.when(c >= 2)
            def _wait_out():
                pltpu.make_async_copy(
                    ob.at[b],
                    out_hbm.at[pl.ds((base + c - 2) * _CHUNK, _CHUNK)],
                    osems[b]).wait()

            compute(c, b)
            pltpu.async_copy(ob.at[b],
                             out_hbm.at[pl.ds((base + c) * _CHUNK, _CHUNK)],
                             osems[b])

            @pl.when(c + 2 < _CPW)
            def _next_gather():
                pltpu.async_copy(emb_hbm.at[sidx_v.at[c + 2]], rv.at[b],
                                 gsems[b])

    for b in range(2):
        c = _CPW - 2 + b
        pltpu.make_async_copy(ob.at[b],
                              out_hbm.at[pl.ds((base + c) * _CHUNK, _CHUNK)],
                              osems[b]).wait()


def kernel(sequence, token_0, token_mask, pe_weight, embeddings):
    seq = sequence.reshape(_NCHUNK, _CHUNK).astype(jnp.int32)
    sidx = jnp.clip(seq - 1, 0, _NUM_ITEMS - 1)
    mesh = plsc.VectorSubcoreMesh(core_axis_name="c", subcore_axis_name="s",
                                  num_cores=_NC, num_subcores=_NS)
    out = pl.kernel(
        _body,
        out_type=jax.ShapeDtypeStruct((_ROWS, _EMBED), jnp.float32),
        mesh=mesh,
        scratch_types=[
            pltpu.VMEM((_CPW, _CHUNK), jnp.int32),
            pltpu.VMEM((_CPW, _CHUNK), jnp.int32),
            pltpu.VMEM((_PE2, _EMBED), jnp.float32),
            pltpu.VMEM((1, _EMBED), jnp.float32),
            pltpu.VMEM((1, _EMBED), jnp.float32),
            pltpu.VMEM((2, _CHUNK, _EMBED), jnp.float32),
            pltpu.VMEM((2, _CHUNK, _EMBED), jnp.float32),
            pltpu.SemaphoreType.DMA,
            pltpu.SemaphoreType.DMA,
            pltpu.SemaphoreType.DMA,
            pltpu.SemaphoreType.DMA,
        ],
        compiler_params=pltpu.CompilerParams(use_tc_tiling_on_sc=False,
                                             needs_layout_passes=False),
    )(seq, sidx, token_0, token_mask, pe_weight, embeddings)
    x = out.reshape(_BATCH, _MAX_LEN, _EMBED)
    return (x, embeddings)


# speedup vs baseline: 5.6933x; 1.9261x over previous
import jax
import jax.numpy as jnp
from jax import lax
from jax.experimental import pallas as pl
from jax.experimental.pallas import tpu as pltpu
from jax.experimental.pallas import tpu_sc as plsc

_EMBED = 128
_MAX_LEN = 200
_NUM_ITEMS = 100000
_BATCH = 1024

_NC, _NS = 2, 16
_NW = _NC * _NS
_ROWS = _BATCH * _MAX_LEN
_CHUNK = 128
_NCHUNK = _ROWS // _CHUNK
_CPW = _NCHUNK // _NW
_NV = _EMBED // 16
_PE2 = _MAX_LEN + _CHUNK


def _body(ridx_hbm, sidx_hbm, t0_hbm, tm_hbm, pe_hbm, emb_hbm, out_hbm,
          ridx_v, sidx_v, pe2_v, pe2_sh, t0_v, tm_v, ob,
          gsem0, gsem1, osem0, osem1):
    gsems = (gsem0, gsem1)
    osems = (osem0, osem1)
    sid = lax.axis_index("s")
    w = sid * _NC + lax.axis_index("c")
    base = w * _CPW
    pltpu.sync_copy(ridx_hbm.at[pl.ds(base, _CPW)], ridx_v)
    pltpu.sync_copy(sidx_hbm.at[pl.ds(base, _CPW)], sidx_v)
    pltpu.sync_copy(pe_hbm, pe2_v.at[pl.ds(0, _MAX_LEN)])
    pltpu.sync_copy(pe_hbm.at[pl.ds(0, _CHUNK)], pe2_v.at[pl.ds(_MAX_LEN, _CHUNK)])
    pltpu.sync_copy(t0_hbm, t0_v)
    pltpu.sync_copy(tm_hbm, tm_v)

    @pl.when(sid == 0)
    def _stage_pe_shared():
        pltpu.sync_copy(pe2_v, pe2_sh)

    plsc.subcore_barrier()
    t0r = [t0_v[0, pl.ds(16 * j, 16)] for j in range(_NV)]
    tmr = [tm_v[0, pl.ds(16 * j, 16)] for j in range(_NV)]

    def start_chunk(c, b):
        tc = (c * _CHUNK) % _MAX_LEN
        pltpu.sync_copy(pe2_sh.at[pl.ds(tc, _CHUNK)], ob.at[b])
        pltpu.async_copy(emb_hbm.at[sidx_v.at[c]], ob.at[b], gsems[b],
                         add=True)

    def fixup(c, b):
        obb = ob.at[b]
        tc = (c * _CHUNK) % _MAX_LEN
        m = None
        for k in range(_NV):
            iv = ridx_v[c, pl.ds(16 * k, 16)]
            mk = (iv == 0) | (iv == _NUM_ITEMS + 1)
            m = mk if m is None else (m | mk)
        any_spec = jnp.any(m)

        @pl.when(any_spec)
        def _slow():
            @pl.loop(0, _CHUNK)
            def _row(l):
                iv = plsc.load_gather(
                    ridx_v,
                    [jnp.full((16,), c, jnp.int32), jnp.full((16,), l, jnp.int32)])
                f0 = iv == 0
                fm = iv == _NUM_ITEMS + 1
                t = tc + l
                for j in range(_NV):
                    p = pe2_v[t, pl.ds(16 * j, 16)]
                    v = obb[l, pl.ds(16 * j, 16)]
                    v = jnp.where(f0, t0r[j] + p, jnp.where(fm, tmr[j] + p, v))
                    obb[l, pl.ds(16 * j, 16)] = v

    for b in range(2):
        start_chunk(b, b)

    @pl.loop(0, _CPW, step=2)
    def _pair(c0):
        for b in range(2):
            c = c0 + b
            pltpu.make_async_copy(emb_hbm.at[sidx_v.at[c]], ob.at[b],
                                  gsems[b]).wait()
            fixup(c, b)
            pltpu.async_copy(ob.at[b],
                             out_hbm.at[pl.ds((base + c) * _CHUNK, _CHUNK)],
                             osems[b])

            @pl.when(c + 2 < _CPW)
            def _next():
                pltpu.make_async_copy(
                    ob.at[b],
                    out_hbm.at[pl.ds((base + c) * _CHUNK, _CHUNK)],
                    osems[b]).wait()
                start_chunk(c + 2, b)

    for b in range(2):
        c = _CPW - 2 + b
        pltpu.make_async_copy(ob.at[b],
                              out_hbm.at[pl.ds((base + c) * _CHUNK, _CHUNK)],
                              osems[b]).wait()


def kernel(sequence, token_0, token_mask, pe_weight, embeddings):
    seq = sequence.reshape(_NCHUNK, _CHUNK).astype(jnp.int32)
    sidx = jnp.clip(seq - 1, 0, _NUM_ITEMS - 1)
    mesh = plsc.VectorSubcoreMesh(core_axis_name="c", subcore_axis_name="s",
                                  num_cores=_NC, num_subcores=_NS)
    out = pl.kernel(
        _body,
        out_type=jax.ShapeDtypeStruct((_ROWS, _EMBED), jnp.float32),
        mesh=mesh,
        scratch_types=[
            pltpu.VMEM((_CPW, _CHUNK), jnp.int32),
            pltpu.VMEM((_CPW, _CHUNK), jnp.int32),
            pltpu.VMEM((_PE2, _EMBED), jnp.float32),
            pltpu.VMEM_SHARED((_PE2, _EMBED), jnp.float32),
            pltpu.VMEM((1, _EMBED), jnp.float32),
            pltpu.VMEM((1, _EMBED), jnp.float32),
            pltpu.VMEM((2, _CHUNK, _EMBED), jnp.float32),
            pltpu.SemaphoreType.DMA,
            pltpu.SemaphoreType.DMA,
            pltpu.SemaphoreType.DMA,
            pltpu.SemaphoreType.DMA,
        ],
        compiler_params=pltpu.CompilerParams(use_tc_tiling_on_sc=False,
                                             needs_layout_passes=False),
    )(seq, sidx, token_0, token_mask, pe_weight, embeddings)
    x = out.reshape(_BATCH, _MAX_LEN, _EMBED)
    return (x, embeddings)


# speedup vs baseline: 6.6017x; 1.1596x over previous
import jax
import jax.numpy as jnp
from jax import lax
from jax.experimental import pallas as pl
from jax.experimental.pallas import tpu as pltpu
from jax.experimental.pallas import tpu_sc as plsc

_EMBED = 128
_MAX_LEN = 200
_NUM_ITEMS = 100000
_BATCH = 1024

_NC, _NS = 2, 16
_NW = _NC * _NS
_ROWS = _BATCH * _MAX_LEN
_CHUNK = 128
_NCHUNK = _ROWS // _CHUNK
_CPW = _NCHUNK // _NW
_NV = _EMBED // 16
_PE2 = _MAX_LEN + _CHUNK
_NB = 5


def _body(ridx_hbm, sidx_hbm, t0_hbm, tm_hbm, pe_hbm, emb_hbm, out_hbm,
          ridx_v, sidx_v, pe2_sh, t0_v, tm_v, e0_v, eN_v, ob,
          gsem0, gsem1, gsem2, gsem3, gsem4,
          osem0, osem1, osem2, osem3, osem4):
    gsems = (gsem0, gsem1, gsem2, gsem3, gsem4)
    osems = (osem0, osem1, osem2, osem3, osem4)
    sid = lax.axis_index("s")
    w = sid * _NC + lax.axis_index("c")
    base = w * _CPW
    pltpu.sync_copy(ridx_hbm.at[pl.ds(base, _CPW)], ridx_v)
    pltpu.sync_copy(sidx_hbm.at[pl.ds(base, _CPW)], sidx_v)
    pltpu.sync_copy(t0_hbm, t0_v)
    pltpu.sync_copy(tm_hbm, tm_v)
    pltpu.sync_copy(emb_hbm.at[pl.ds(0, 1)], e0_v)
    pltpu.sync_copy(emb_hbm.at[pl.ds(_NUM_ITEMS - 1, 1)], eN_v)

    @pl.when(sid == 0)
    def _stage_pe_shared():
        bounce = ob.at[_NB - 1]
        pltpu.sync_copy(pe_hbm.at[pl.ds(0, _CHUNK)], bounce)
        pltpu.sync_copy(bounce, pe2_sh.at[pl.ds(0, _CHUNK)])
        pltpu.sync_copy(bounce, pe2_sh.at[pl.ds(_MAX_LEN, _CHUNK)])
        rest = _MAX_LEN - _CHUNK
        pltpu.sync_copy(pe_hbm.at[pl.ds(_CHUNK, rest)],
                        bounce.at[pl.ds(0, rest)])
        pltpu.sync_copy(bounce.at[pl.ds(0, rest)],
                        pe2_sh.at[pl.ds(_CHUNK, rest)])

    plsc.subcore_barrier()

    d0r = [t0_v[0, pl.ds(16 * j, 16)] - e0_v[0, pl.ds(16 * j, 16)]
           for j in range(_NV)]
    dmr = [tm_v[0, pl.ds(16 * j, 16)] - eN_v[0, pl.ds(16 * j, 16)]
           for j in range(_NV)]
    zf = jnp.zeros((16,), jnp.float32)

    def start_chunk(c, b):
        tc = (c * _CHUNK) % _MAX_LEN
        pltpu.sync_copy(pe2_sh.at[pl.ds(tc, _CHUNK)], ob.at[b])
        pltpu.async_copy(emb_hbm.at[sidx_v.at[c]], ob.at[b], gsems[b],
                         add=True)

    def fixup(c, b):
        obb = ob.at[b]
        m = None
        for k in range(_NV):
            iv = ridx_v[c, pl.ds(16 * k, 16)]
            mk = (iv == 0) | (iv == _NUM_ITEMS + 1)
            m = mk if m is None else (m | mk)
        any_spec = jnp.any(m)

        @pl.when(any_spec)
        def _slow():
            @pl.loop(0, _CHUNK)
            def _row(l):
                iv = plsc.load_gather(
                    ridx_v,
                    [jnp.full((16,), c, jnp.int32), jnp.full((16,), l, jnp.int32)])
                f0 = iv == 0
                fm = iv == _NUM_ITEMS + 1

                @pl.when(jnp.any(f0 | fm))
                def _patch():
                    for j in range(_NV):
                        v = obb[l, pl.ds(16 * j, 16)]
                        v = v + jnp.where(f0, d0r[j], jnp.where(fm, dmr[j], zf))
                        obb[l, pl.ds(16 * j, 16)] = v

    def out_slice(c):
        return out_hbm.at[pl.ds((base + c) * _CHUNK, _CHUNK)]

    for j in range(_NB - 1):
        start_chunk(j, j)

    @pl.loop(0, _CPW, step=_NB)
    def _ring(c0):
        for j in range(_NB):
            c = c0 + j
            pltpu.make_async_copy(emb_hbm.at[sidx_v.at[c]], ob.at[j],
                                  gsems[j]).wait()
            fixup(c, j)
            pltpu.async_copy(ob.at[j], out_slice(c), osems[j])

            d = c + _NB - 1
            bd = (j + _NB - 1) % _NB

            @pl.when(d < _CPW)
            def _issue():
                @pl.when(c >= 1)
                def _wait_prev_out():
                    pltpu.make_async_copy(ob.at[bd], out_slice(c - 1),
                                          osems[bd]).wait()

                start_chunk(d, bd)

    for j in range(_NB):
        c = _CPW - _NB + j
        pltpu.make_async_copy(ob.at[j], out_slice(c), osems[j]).wait()


def kernel(sequence, token_0, token_mask, pe_weight, embeddings):
    seq = sequence.reshape(_NCHUNK, _CHUNK).astype(jnp.int32)
    sidx = jnp.clip(seq - 1, 0, _NUM_ITEMS - 1)
    mesh = plsc.VectorSubcoreMesh(core_axis_name="c", subcore_axis_name="s",
                                  num_cores=_NC, num_subcores=_NS)
    out = pl.kernel(
        _body,
        out_type=jax.ShapeDtypeStruct((_ROWS, _EMBED), jnp.float32),
        mesh=mesh,
        scratch_types=[
            pltpu.VMEM((_CPW, _CHUNK), jnp.int32),
            pltpu.VMEM((_CPW, _CHUNK), jnp.int32),
            pltpu.VMEM_SHARED((_PE2, _EMBED), jnp.float32),
            pltpu.VMEM((1, _EMBED), jnp.float32),
            pltpu.VMEM((1, _EMBED), jnp.float32),
            pltpu.VMEM((1, _EMBED), jnp.float32),
            pltpu.VMEM((1, _EMBED), jnp.float32),
            pltpu.VMEM((_NB, _CHUNK, _EMBED), jnp.float32),
        ] + [pltpu.SemaphoreType.DMA] * (2 * _NB),
        compiler_params=pltpu.CompilerParams(use_tc_tiling_on_sc=False,
                                             needs_layout_passes=False),
    )(seq, sidx, token_0, token_mask, pe_weight, embeddings)
    x = out.reshape(_BATCH, _MAX_LEN, _EMBED)
    return (x, embeddings)


# speedup vs baseline: 6.6145x; 1.0019x over previous
import jax
import jax.numpy as jnp
from jax import lax
from jax.experimental import pallas as pl
from jax.experimental.pallas import tpu as pltpu
from jax.experimental.pallas import tpu_sc as plsc

_EMBED = 128
_MAX_LEN = 200
_NUM_ITEMS = 100000
_BATCH = 1024

_NC, _NS = 2, 16
_NW = _NC * _NS
_ROWS = _BATCH * _MAX_LEN
_CHUNK = 128
_NCHUNK = _ROWS // _CHUNK
_CPW = _NCHUNK // _NW
_NV = _EMBED // 16
_PE2 = _MAX_LEN + _CHUNK
_NB = 5


def _body(ridx_hbm, sidx_hbm, t0_hbm, tm_hbm, pe_hbm, emb_hbm, out_hbm,
          ridx_v, sidx_v, pe2_sh, t0_v, tm_v, e0_v, eN_v, ob,
          gsem0, gsem1, gsem2, gsem3, gsem4,
          osem0, osem1, osem2, osem3, osem4):
    gsems = (gsem0, gsem1, gsem2, gsem3, gsem4)
    osems = (osem0, osem1, osem2, osem3, osem4)
    sid = lax.axis_index("s")
    w = sid * _NC + lax.axis_index("c")
    base = w * _CPW
    pltpu.sync_copy(ridx_hbm.at[pl.ds(base, _CPW)], ridx_v)
    pltpu.sync_copy(sidx_hbm.at[pl.ds(base, _CPW)], sidx_v)
    pltpu.sync_copy(t0_hbm, t0_v)
    pltpu.sync_copy(tm_hbm, tm_v)
    pltpu.sync_copy(emb_hbm.at[pl.ds(0, 1)], e0_v)
    pltpu.sync_copy(emb_hbm.at[pl.ds(_NUM_ITEMS - 1, 1)], eN_v)

    @pl.when(sid == 0)
    def _stage_pe_shared():
        bounce = ob.at[_NB - 1]
        pltpu.sync_copy(pe_hbm.at[pl.ds(0, _CHUNK)], bounce)
        pltpu.sync_copy(bounce, pe2_sh.at[pl.ds(0, _CHUNK)])
        pltpu.sync_copy(bounce, pe2_sh.at[pl.ds(_MAX_LEN, _CHUNK)])
        rest = _MAX_LEN - _CHUNK
        pltpu.sync_copy(pe_hbm.at[pl.ds(_CHUNK, rest)],
                        bounce.at[pl.ds(0, rest)])
        pltpu.sync_copy(bounce.at[pl.ds(0, rest)],
                        pe2_sh.at[pl.ds(_CHUNK, rest)])

    plsc.subcore_barrier()

    d0r = [t0_v[0, pl.ds(16 * j, 16)] - e0_v[0, pl.ds(16 * j, 16)]
           for j in range(_NV)]
    dmr = [tm_v[0, pl.ds(16 * j, 16)] - eN_v[0, pl.ds(16 * j, 16)]
           for j in range(_NV)]
    zf = jnp.zeros((16,), jnp.float32)

    def start_chunk(c, b):
        tc = (c * _CHUNK) % _MAX_LEN
        pltpu.sync_copy(pe2_sh.at[pl.ds(tc, _CHUNK)], ob.at[b])
        pltpu.async_copy(emb_hbm.at[sidx_v.at[c]], ob.at[b], gsems[b],
                         add=True)

    def fixup(c, b):
        obb = ob.at[b]
        m = None
        for k in range(_NV):
            iv = ridx_v[c, pl.ds(16 * k, 16)]
            mk = (iv == 0) | (iv == _NUM_ITEMS + 1)
            m = mk if m is None else (m | mk)
        any_spec = jnp.any(m)

        @pl.when(any_spec)
        def _slow():
            @pl.loop(0, _CHUNK)
            def _row(l):
                iv = plsc.load_gather(
                    ridx_v,
                    [jnp.full((16,), c, jnp.int32), jnp.full((16,), l, jnp.int32)])
                f0 = iv == 0
                fm = iv == _NUM_ITEMS + 1

                @pl.when(jnp.any(f0 | fm))
                def _patch():
                    for j in range(_NV):
                        v = obb[l, pl.ds(16 * j, 16)]
                        v = v + jnp.where(f0, d0r[j], jnp.where(fm, dmr[j], zf))
                        obb[l, pl.ds(16 * j, 16)] = v

    def out_slice(c):
        return out_hbm.at[pl.ds((base + c) * _CHUNK, _CHUNK)]

    for j in range(_NB - 1):
        start_chunk(j, j)

    @pl.loop(0, _CPW, step=_NB)
    def _ring(c0):
        for j in range(_NB):
            c = c0 + j
            pltpu.make_async_copy(emb_hbm.at[sidx_v.at[c]], ob.at[j],
                                  gsems[j]).wait()
            fixup(c, j)
            pltpu.async_copy(ob.at[j], out_slice(c), osems[j])

            d = c + _NB - 1
            bd = (j + _NB - 1) % _NB

            @pl.when(d < _CPW)
            def _issue():
                @pl.when(c >= 1)
                def _wait_prev_out():
                    pltpu.make_async_copy(ob.at[bd], out_slice(c - 1),
                                          osems[bd]).wait()

                start_chunk(d, bd)

    for j in range(_NB):
        c = _CPW - _NB + j
        pltpu.make_async_copy(ob.at[j], out_slice(c), osems[j]).wait()


def kernel(sequence, token_0, token_mask, pe_weight, embeddings):
    emb_out = jnp.copy(embeddings)
    seq = sequence.reshape(_NCHUNK, _CHUNK).astype(jnp.int32)
    sidx = jnp.clip(seq - 1, 0, _NUM_ITEMS - 1)
    mesh = plsc.VectorSubcoreMesh(core_axis_name="c", subcore_axis_name="s",
                                  num_cores=_NC, num_subcores=_NS)
    out = pl.kernel(
        _body,
        out_type=jax.ShapeDtypeStruct((_ROWS, _EMBED), jnp.float32),
        mesh=mesh,
        scratch_types=[
            pltpu.VMEM((_CPW, _CHUNK), jnp.int32),
            pltpu.VMEM((_CPW, _CHUNK), jnp.int32),
            pltpu.VMEM_SHARED((_PE2, _EMBED), jnp.float32),
            pltpu.VMEM((1, _EMBED), jnp.float32),
            pltpu.VMEM((1, _EMBED), jnp.float32),
            pltpu.VMEM((1, _EMBED), jnp.float32),
            pltpu.VMEM((1, _EMBED), jnp.float32),
            pltpu.VMEM((_NB, _CHUNK, _EMBED), jnp.float32),
        ] + [pltpu.SemaphoreType.DMA] * (2 * _NB),
        compiler_params=pltpu.CompilerParams(use_tc_tiling_on_sc=False,
                                             needs_layout_passes=False),
    )(seq, sidx, token_0, token_mask, pe_weight, embeddings)
    x = out.reshape(_BATCH, _MAX_LEN, _EMBED)
    return (x, emb_out)
